# Initial kernel scaffold; baseline (speedup 1.0000x reference)
import jax
import jax.numpy as jnp
from jax.experimental import pallas as pl


def kernel(feature_map, bbox_yx, sample_association):
    raise NotImplementedError("write your pallas kernel here")


# speedup vs baseline: 128.3768x; 128.3768x over previous
import functools

import jax
import jax.numpy as jnp
from jax import lax
from jax.experimental import pallas as pl
from jax.experimental.pallas import tpu as pltpu
from jax.experimental.pallas import tpu_sc as plsc

BOX_H = 16
BOX_W = 16


def _iota16():
    return lax.broadcasted_iota(jnp.int32, (16,), 0)


def kernel(feature_map, bbox_yx, sample_association):
    B, C, H, W = feature_map.shape
    N = bbox_yx.shape[0]
    assoc = sample_association.astype(jnp.int32)
    bbox_flat = bbox_yx.astype(jnp.int32).reshape(-1)

    NW = 32
    npw = N // NW

    mesh = plsc.VectorSubcoreMesh(core_axis_name="c", subcore_axis_name="s")

    @functools.partial(
        pl.kernel,
        mesh=mesh,
        compiler_params=pltpu.CompilerParams(use_tc_tiling_on_sc=False),
        out_type=(
            jax.ShapeDtypeStruct((N, C, BOX_H, BOX_W), jnp.float32),
            jax.ShapeDtypeStruct((N * 4,), jnp.int32),
            jax.ShapeDtypeStruct((B,), jnp.int32),
        ),
        scratch_types=[
            pltpu.VMEM((npw * 2 + 16,), jnp.int32),
            pltpu.VMEM((N + 16,), jnp.int32),
            pltpu.VMEM((npw * 4,), jnp.int32),
            pltpu.VMEM((16,), jnp.int32),
            pltpu.VMEM((C, BOX_H, BOX_W + 8), jnp.float32),
            pltpu.SemaphoreType.DMA,
        ],
    )
    def roi_cut(fm, bbox_h, assoc_h, out_h, bt_h, cnt_h,
                bbox_v, assoc_v, bt_v, cnt_v, stage_v, sem):
        cid = lax.axis_index("c")
        sid = lax.axis_index("s")
        wid = sid * 2 + cid
        base = wid * npw

        pltpu.sync_copy(bbox_h.at[pl.ds(base * 2, npw * 2)],
                        bbox_v.at[pl.ds(0, npw * 2)])
        pltpu.sync_copy(assoc_h, assoc_v.at[pl.ds(0, N)])

        def box_body(i, _):
            bvec = bbox_v[pl.ds(2 * i, 16)]
            avec = assoc_v[pl.ds(base + i, 16)]
            y = bvec[0]
            x = bvec[1]
            b = avec[0]
            y0 = jnp.minimum(jnp.maximum(y, 0), H - BOX_H)
            x0 = jnp.minimum(jnp.maximum(x, 0), W - BOX_W)
            x0a = pl.multiple_of((x0 // 8) * 8, 8)
            r = x0 - x0a
            pltpu.sync_copy(
                fm.at[b, :, pl.ds(y0, BOX_H), pl.ds(x0a, BOX_W + 8)],
                stage_v,
            )

            def shift_body(c, _):
                for row in range(BOX_H):
                    v = stage_v[c, row, pl.ds(r, BOX_W)]
                    stage_v[c, row, pl.ds(0, BOX_W)] = v
                return 0

            lax.fori_loop(0, jnp.where(r > 0, C, 0), shift_body, 0)
            pltpu.sync_copy(
                stage_v.at[:, :, pl.ds(0, BOX_W)],
                out_h.at[base + i],
            )
            return 0

        lax.fori_loop(0, npw, box_body, 0)

        lanes = _iota16()
        comp = lanes & 3
        quad = lanes >> 2
        for k in range(npw // 4):
            chunk = lanes * 0
            bvecs = bbox_v[pl.ds(8 * k, 16)]
            for q in range(4):
                y = bvecs[2 * q]
                x = bvecs[2 * q + 1]
                y0 = jnp.minimum(jnp.maximum(y, 0), H - BOX_H)
                x0 = jnp.minimum(jnp.maximum(x, 0), W - BOX_W)
                bvec_q = jnp.where(
                    comp == 0, y0,
                    jnp.where(comp == 1, x0,
                              jnp.where(comp == 2, y0 + BOX_H, x0 + BOX_W)))
                chunk = jnp.where(quad == q, bvec_q, chunk)
            bt_v[pl.ds(16 * k, 16)] = chunk
        pltpu.sync_copy(bt_v, bt_h.at[pl.ds(base * 4, npw * 4)])
        @pl.when(wid == 0)
        def _():
            def upper_bound(bb):
                def step(t, lohi):
                    lo, hi = lohi
                    mid = (lo + hi) >> 1
                    v = assoc_v[pl.ds(mid, 16)][0]
                    gt = v > bb
                    return (jnp.where(gt, lo, mid + 1), jnp.where(gt, mid, hi))

                lo, _hi = lax.fori_loop(0, 10, step, (0, N))
                return lo

            cnt = lanes * 0
            prev = 0
            for bb in range(B):
                ub = upper_bound(bb)
                cnt = jnp.where(lanes == bb, ub - prev, cnt)
                prev = ub
            cnt_v[...] = cnt
            pltpu.sync_copy(cnt_v.at[pl.ds(0, B)], cnt_h)

    box_features, bt_flat, counts = roi_cut(feature_map, bbox_flat, assoc)
    return (box_features, (bt_flat.reshape(N, 4), counts, (H, W)))


# speedup vs baseline: 182.2355x; 1.4195x over previous
import functools

import jax
import jax.numpy as jnp
from jax import lax
from jax.experimental import pallas as pl
from jax.experimental.pallas import tpu as pltpu
from jax.experimental.pallas import tpu_sc as plsc

BOX_H = 16
BOX_W = 16


def _iota16():
    return lax.broadcasted_iota(jnp.int32, (16,), 0)


def kernel(feature_map, bbox_yx, sample_association):
    B, C, H, W = feature_map.shape
    N = bbox_yx.shape[0]
    assoc = sample_association.astype(jnp.int32)
    bbox_flat = bbox_yx.astype(jnp.int32).reshape(-1)

    NW = 32
    npw = N // NW

    mesh = plsc.VectorSubcoreMesh(core_axis_name="c", subcore_axis_name="s")

    @functools.partial(
        pl.kernel,
        mesh=mesh,
        compiler_params=pltpu.CompilerParams(use_tc_tiling_on_sc=False),
        out_type=(
            jax.ShapeDtypeStruct((N, C, BOX_H, BOX_W), jnp.float32),
            jax.ShapeDtypeStruct((N * 4,), jnp.int32),
            jax.ShapeDtypeStruct((B,), jnp.int32),
        ),
        scratch_types=[
            pltpu.VMEM((npw * 2 + 16,), jnp.int32),
            pltpu.VMEM((N + 16,), jnp.int32),
            pltpu.VMEM((npw * 4,), jnp.int32),
            pltpu.VMEM((16,), jnp.int32),
            pltpu.VMEM((C // 4, BOX_H, BOX_W + 8), jnp.float32),
            pltpu.VMEM((C // 4, BOX_H, BOX_W + 8), jnp.float32),
            pltpu.VMEM((C // 4, BOX_H, BOX_W), jnp.float32),
            pltpu.VMEM((C // 4, BOX_H, BOX_W), jnp.float32),
            pltpu.SemaphoreType.DMA,
            pltpu.SemaphoreType.DMA,
        ],
    )
    def roi_cut(fm, bbox_h, assoc_h, out_h, bt_h, cnt_h,
                bbox_v, assoc_v, bt_v, cnt_v, in0, in1, ou0, ou1,
                sem_in, sem_out):
        cid = lax.axis_index("c")
        sid = lax.axis_index("s")
        wid = sid * 2 + cid
        base = wid * npw

        pltpu.sync_copy(bbox_h.at[pl.ds(base * 2, npw * 2)],
                        bbox_v.at[pl.ds(0, npw * 2)])
        pltpu.sync_copy(assoc_h, assoc_v.at[pl.ds(0, N)])

        NCH = 4
        CH = C // NCH
        T = npw * NCH

        def params(t):
            i = t >> 2
            k = t & 3
            bvec = bbox_v[pl.ds(2 * i, 16)]
            avec = assoc_v[pl.ds(base + i, 16)]
            y0 = jnp.minimum(jnp.maximum(bvec[0], 0), H - BOX_H)
            x0 = jnp.minimum(jnp.maximum(bvec[1], 0), W - BOX_W)
            b = avec[0]
            x0a = pl.multiple_of((x0 >> 3) << 3, 8)
            r = x0 - x0a
            return i, k, b, y0, x0a, r

        def start_in(t, inb):
            i, k, b, y0, x0a, r = params(t)
            pltpu.async_copy(
                fm.at[b, pl.ds(k * CH, CH), pl.ds(y0, BOX_H),
                      pl.ds(x0a, BOX_W + 8)],
                inb, sem_in)

        def process(t, inb, oub):
            pltpu.make_async_copy(
                fm.at[0, pl.ds(0, CH), pl.ds(0, BOX_H), pl.ds(0, BOX_W + 8)],
                inb, sem_in).wait()

            @pl.when(t >= 2)
            def _():
                pltpu.make_async_copy(
                    oub, out_h.at[0, pl.ds(0, CH)], sem_out).wait()

            i, k, b, y0, x0a, r = params(t)

            @plsc.parallel_loop(0, CH, unroll=2)
            def _(c):
                for row in range(BOX_H):
                    oub[c, row] = inb[c, row, pl.ds(r, BOX_W)]

            @pl.when(t + 2 < T)
            def _():
                start_in(t + 2, inb)

            pltpu.async_copy(
                oub, out_h.at[base + i, pl.ds(k * CH, CH)], sem_out)

        start_in(0, in0)
        start_in(1, in1)

        def pair_body(u, _):
            process(2 * u, in0, ou0)
            process(2 * u + 1, in1, ou1)
            return 0

        lax.fori_loop(0, T // 2, pair_body, 0)
        pltpu.make_async_copy(ou0, out_h.at[0, pl.ds(0, CH)], sem_out).wait()
        pltpu.make_async_copy(ou1, out_h.at[0, pl.ds(0, CH)], sem_out).wait()

        lanes = _iota16()
        comp = lanes & 3
        quad = lanes >> 2
        for k in range(npw // 4):
            chunk = lanes * 0
            bvecs = bbox_v[pl.ds(8 * k, 16)]
            for q in range(4):
                y = bvecs[2 * q]
                x = bvecs[2 * q + 1]
                y0 = jnp.minimum(jnp.maximum(y, 0), H - BOX_H)
                x0 = jnp.minimum(jnp.maximum(x, 0), W - BOX_W)
                bvec_q = jnp.where(
                    comp == 0, y0,
                    jnp.where(comp == 1, x0,
                              jnp.where(comp == 2, y0 + BOX_H, x0 + BOX_W)))
                chunk = jnp.where(quad == q, bvec_q, chunk)
            bt_v[pl.ds(16 * k, 16)] = chunk
        pltpu.sync_copy(bt_v, bt_h.at[pl.ds(base * 4, npw * 4)])
        @pl.when(wid == 0)
        def _():
            def upper_bound(bb):
                def step(t, lohi):
                    lo, hi = lohi
                    mid = (lo + hi) >> 1
                    v = assoc_v[pl.ds(mid, 16)][0]
                    gt = v > bb
                    return (jnp.where(gt, lo, mid + 1), jnp.where(gt, mid, hi))

                lo, _hi = lax.fori_loop(0, 10, step, (0, N))
                return lo

            cnt = lanes * 0
            prev = 0
            for bb in range(B):
                ub = upper_bound(bb)
                cnt = jnp.where(lanes == bb, ub - prev, cnt)
                prev = ub
            cnt_v[...] = cnt
            pltpu.sync_copy(cnt_v.at[pl.ds(0, B)], cnt_h)

    box_features, bt_flat, counts = roi_cut(feature_map, bbox_flat, assoc)
    return (box_features, (bt_flat.reshape(N, 4), counts, (H, W)))


# speedup vs baseline: 1680.6776x; 9.2226x over previous
import functools

import jax
import jax.numpy as jnp
from jax import lax
from jax.experimental import pallas as pl
from jax.experimental.pallas import tpu as pltpu
from jax.experimental.pallas import tpu_sc as plsc

BOX_H = 16
BOX_W = 16


def _iota16():
    return lax.broadcasted_iota(jnp.int32, (16,), 0)


def kernel(feature_map, bbox_yx, sample_association):
    B, C, H, W = feature_map.shape
    N = bbox_yx.shape[0]
    assoc = sample_association.astype(jnp.int32)
    bbox_flat = bbox_yx.astype(jnp.int32).reshape(-1)

    NW = 32
    npw = N // NW

    mesh = plsc.VectorSubcoreMesh(core_axis_name="c", subcore_axis_name="s")

    @functools.partial(
        pl.kernel,
        mesh=mesh,
        compiler_params=pltpu.CompilerParams(use_tc_tiling_on_sc=False, needs_layout_passes=False),
        out_type=(
            jax.ShapeDtypeStruct((N, BOX_H, BOX_W // 8, C // 128, 8, 128),
                                 jnp.float32),
            jax.ShapeDtypeStruct((N * 4,), jnp.int32),
            jax.ShapeDtypeStruct((B,), jnp.int32),
        ),
        scratch_types=[
            pltpu.VMEM((npw * 2 + 16,), jnp.int32),
            pltpu.VMEM((N + 16,), jnp.int32),
            pltpu.VMEM((npw * 4,), jnp.int32),
            pltpu.VMEM((16,), jnp.int32),
            pltpu.VMEM((C // 4, BOX_H, BOX_W + 8), jnp.float32),
            pltpu.VMEM((C // 4, BOX_H, BOX_W + 8), jnp.float32),
            pltpu.VMEM((BOX_H, BOX_W // 8, 8, 129), jnp.float32),
            pltpu.VMEM((BOX_H, BOX_W // 8, 8, 129), jnp.float32),
            pltpu.SemaphoreType.DMA,
            pltpu.SemaphoreType.DMA,
        ],
    )
    def roi_cut(fm, bbox_h, assoc_h, out_h, bt_h, cnt_h,
                bbox_v, assoc_v, bt_v, cnt_v, in0, in1, tb0, tb1,
                sem_in, sem_out):
        cid = lax.axis_index("c")
        sid = lax.axis_index("s")
        wid = sid * 2 + cid
        base = wid * npw

        pltpu.sync_copy(bbox_h.at[pl.ds(base * 2, npw * 2)],
                        bbox_v.at[pl.ds(0, npw * 2)])
        pltpu.sync_copy(assoc_h, assoc_v.at[pl.ds(0, N)])

        NCH = 4
        CH = C // NCH
        T = npw * NCH

        lanes16 = _iota16()
        jt_idx = lanes16 >> 3
        js_idx = lanes16 & 7

        def params(t):
            i = t >> 2
            k = t & 3
            bvec = bbox_v[pl.ds(2 * i, 16)]
            avec = assoc_v[pl.ds(base + i, 16)]
            y0 = jnp.minimum(jnp.maximum(bvec[0], 0), H - BOX_H)
            x0 = jnp.minimum(jnp.maximum(bvec[1], 0), W - BOX_W)
            b = avec[0]
            x0a = pl.multiple_of((x0 >> 3) << 3, 8)
            r = x0 - x0a
            return i, k, b, y0, x0a, r

        def start_in(t, inb):
            i, k, b, y0, x0a, r = params(t)
            pltpu.async_copy(
                fm.at[b, pl.ds(k * CH, CH), pl.ds(y0, BOX_H),
                      pl.ds(x0a, BOX_W + 8)],
                inb, sem_in)

        def process(t, kmod, inb, tb):
            ct = kmod >> 1
            cs0 = (kmod & 1) * CH

            pltpu.make_async_copy(
                fm.at[0, pl.ds(0, CH), pl.ds(0, BOX_H), pl.ds(0, BOX_W + 8)],
                inb, sem_in).wait()

            if kmod & 1 == 0:
                @pl.when(t >= 4)
                def _():
                    pltpu.make_async_copy(
                        tb.at[:, :, :, pl.ds(0, 128)],
                        out_h.at[0, :, :, 0], sem_out).wait()

            i, k, b, y0, x0a, r = params(t)

            @plsc.parallel_loop(0, CH, unroll=2)
            def _(c):
                cs_idx = lanes16 * 0 + (cs0 + c)
                for row in range(BOX_H):
                    v = inb[c, row, pl.ds(r, BOX_W)]
                    plsc.store_scatter(
                        tb, [lanes16 * 0 + row, jt_idx, js_idx, cs_idx], v)

            @pl.when(t + 2 < T)
            def _():
                start_in(t + 2, inb)

            if kmod & 1 == 1:
                pltpu.async_copy(
                    tb.at[:, :, :, pl.ds(0, 128)],
                    out_h.at[base + i, :, :, ct], sem_out)

        start_in(0, in0)
        start_in(1, in1)

        def quad_body(u, _):
            t0 = 4 * u
            process(t0, 0, in0, tb0)
            process(t0 + 1, 1, in1, tb0)
            process(t0 + 2, 2, in0, tb1)
            process(t0 + 3, 3, in1, tb1)
            return 0

        lax.fori_loop(0, T // 4, quad_body, 0)
        pltpu.make_async_copy(tb0.at[:, :, :, pl.ds(0, 128)],
                              out_h.at[0, :, :, 0], sem_out).wait()
        pltpu.make_async_copy(tb1.at[:, :, :, pl.ds(0, 128)],
                              out_h.at[0, :, :, 0], sem_out).wait()

        lanes = _iota16()
        comp = lanes & 3
        quad = lanes >> 2
        for k in range(npw // 4):
            chunk = lanes * 0
            bvecs = bbox_v[pl.ds(8 * k, 16)]
            for q in range(4):
                y = bvecs[2 * q]
                x = bvecs[2 * q + 1]
                y0 = jnp.minimum(jnp.maximum(y, 0), H - BOX_H)
                x0 = jnp.minimum(jnp.maximum(x, 0), W - BOX_W)
                bvec_q = jnp.where(
                    comp == 0, y0,
                    jnp.where(comp == 1, x0,
                              jnp.where(comp == 2, y0 + BOX_H, x0 + BOX_W)))
                chunk = jnp.where(quad == q, bvec_q, chunk)
            bt_v[pl.ds(16 * k, 16)] = chunk
        pltpu.sync_copy(bt_v, bt_h.at[pl.ds(base * 4, npw * 4)])
        @pl.when(wid == 0)
        def _():
            def upper_bound(bb):
                def step(t, lohi):
                    lo, hi = lohi
                    mid = (lo + hi) >> 1
                    v = assoc_v[pl.ds(mid, 16)][0]
                    gt = v > bb
                    return (jnp.where(gt, lo, mid + 1), jnp.where(gt, mid, hi))

                lo, _hi = lax.fori_loop(0, 10, step, (0, N))
                return lo

            cnt = lanes * 0
            prev = 0
            for bb in range(B):
                ub = upper_bound(bb)
                cnt = jnp.where(lanes == bb, ub - prev, cnt)
                prev = ub
            cnt_v[...] = cnt
            pltpu.sync_copy(cnt_v.at[pl.ds(0, B)], cnt_h)

    out6, bt_flat, counts = roi_cut(feature_map, bbox_flat, assoc)
    box_features = out6.transpose(0, 3, 5, 1, 2, 4).reshape(N, C, BOX_H, BOX_W)
    return (box_features, (bt_flat.reshape(N, 4), counts, (H, W)))
